# Initial kernel scaffold; baseline (speedup 1.0000x reference)
import functools

import jax
import jax.numpy as jnp
from jax import lax
from jax.experimental import pallas as pl
from jax.experimental.pallas import tpu as pltpu
from jax.experimental.pallas import tpu_sc as plsc

N_NODES = 10000
N_EDGES = 320000
D = 128
BOUND = 5.0

NC = 2
NS = 16
NW = NC * NS
E_W = N_EDGES // NW
CH = 125
NCHUNK = E_W // CH
ROWS_PER_TILE = N_NODES // NS
ZCH = ROWS_PER_TILE // CH


def _segment_sum_sc(x, src, dst, zeros):
  mesh = plsc.VectorSubcoreMesh(core_axis_name="c", subcore_axis_name="s")

  @functools.partial(
      pl.kernel,
      out_type=jax.ShapeDtypeStruct((NC, N_NODES, D), jnp.float32),
      mesh=mesh,
      scratch_types=[
          pltpu.VMEM((NCHUNK, CH), jnp.int32),
          pltpu.VMEM((NCHUNK, CH), jnp.int32),
          pltpu.VMEM((CH, D), jnp.float32),
          pltpu.VMEM((CH, D), jnp.float32),
          pltpu.VMEM_SHARED((N_NODES, D), jnp.float32),
          pltpu.SemaphoreType.DMA,
          pltpu.SemaphoreType.DMA,
      ],
  )
  def kern(x_hbm, src_hbm, dst_hbm, zeros_hbm, out_hbm,
           src_v, dst_v, buf0, buf1, agg_sh, sem0, sem1):
    c = lax.axis_index("c")
    s = lax.axis_index("s")
    wid = s * NC + c
    pltpu.sync_copy(src_hbm.at[wid], src_v)
    pltpu.sync_copy(dst_hbm.at[wid], dst_v)
    pltpu.sync_copy(zeros_hbm, buf0)
    row0 = s * ROWS_PER_TILE
    for r in range(ZCH):
      pltpu.sync_copy(buf0, agg_sh.at[pl.ds(row0 + r * CH, CH)])
    plsc.subcore_barrier()

    pltpu.async_copy(x_hbm.at[src_v.at[0]], buf0, sem0)
    pltpu.async_copy(x_hbm.at[src_v.at[1]], buf1, sem1)

    def body(g2, carry):
      c0 = g2 * 2
      c1 = c0 + 1
      pltpu.make_async_copy(x_hbm.at[src_v.at[c0]], buf0, sem0).wait()
      pltpu.sync_copy(buf0, agg_sh.at[dst_v.at[c0]], add=True)

      @pl.when(c0 + 2 < NCHUNK)
      def _():
        pltpu.async_copy(x_hbm.at[src_v.at[c0 + 2]], buf0, sem0)

      pltpu.make_async_copy(x_hbm.at[src_v.at[c1]], buf1, sem1).wait()
      pltpu.sync_copy(buf1, agg_sh.at[dst_v.at[c1]], add=True)

      @pl.when(c1 + 2 < NCHUNK)
      def _():
        pltpu.async_copy(x_hbm.at[src_v.at[c1 + 2]], buf1, sem1)

      return carry

    lax.fori_loop(0, NCHUNK // 2, body, 0)
    plsc.subcore_barrier()
    for r in range(ZCH):
      sl = pl.ds(row0 + r * CH, CH)
      pltpu.sync_copy(agg_sh.at[sl], buf0)
      pltpu.sync_copy(buf0, out_hbm.at[c].at[sl])

  return kern(x, src, dst, zeros)


BLK = 1000


def _softmax(z):
  z = z - jnp.max(z, axis=-1, keepdims=True)
  e = jnp.exp(z)
  return e / jnp.sum(e, axis=-1, keepdims=True)


def _tc_input(x, w):
  def body(x_ref, w_ref, o_ref):
    z = jnp.dot(x_ref[...], w_ref[...], preferred_element_type=jnp.float32)
    o_ref[...] = _softmax(z)

  return pl.pallas_call(
      body,
      grid=(N_NODES // BLK,),
      in_specs=[
          pl.BlockSpec((BLK, D), lambda i: (i, 0)),
          pl.BlockSpec((D, D), lambda i: (0, 0)),
      ],
      out_specs=pl.BlockSpec((BLK, D), lambda i: (i, 0)),
      out_shape=jax.ShapeDtypeStruct((N_NODES, D), jnp.float32),
  )(x, w)


def _tc_layer(agg, h, w_a, w_h):
  def body(a_ref, h_ref, wa_ref, wh_ref, o_ref):
    a = jnp.clip(a_ref[0] + a_ref[1], 0.0, BOUND)
    z = jnp.dot(a, wa_ref[...], preferred_element_type=jnp.float32)
    z = z + jnp.dot(h_ref[...], wh_ref[...], preferred_element_type=jnp.float32)
    o_ref[...] = _softmax(z)

  return pl.pallas_call(
      body,
      grid=(N_NODES // BLK,),
      in_specs=[
          pl.BlockSpec((NC, BLK, D), lambda i: (0, i, 0)),
          pl.BlockSpec((BLK, D), lambda i: (i, 0)),
          pl.BlockSpec((D, D), lambda i: (0, 0)),
          pl.BlockSpec((D, D), lambda i: (0, 0)),
      ],
      out_specs=pl.BlockSpec((BLK, D), lambda i: (i, 0)),
      out_shape=jax.ShapeDtypeStruct((N_NODES, D), jnp.float32),
  )(agg, h, w_a, w_h)


def _tc_layer_pool(agg, h, w_a, w_h, w_pool):
  def body(a_ref, h_ref, wa_ref, wh_ref, wp_ref, o_ref):
    a = jnp.clip(a_ref[0] + a_ref[1], 0.0, BOUND)
    z = jnp.dot(a, wa_ref[...], preferred_element_type=jnp.float32)
    z = z + jnp.dot(h_ref[...], wh_ref[...], preferred_element_type=jnp.float32)
    h1 = _softmax(z)
    o_ref[...] = _softmax(
        jnp.dot(h1, wp_ref[...], preferred_element_type=jnp.float32))

  return pl.pallas_call(
      body,
      grid=(N_NODES // BLK,),
      in_specs=[
          pl.BlockSpec((NC, BLK, D), lambda i: (0, i, 0)),
          pl.BlockSpec((BLK, D), lambda i: (i, 0)),
          pl.BlockSpec((D, D), lambda i: (0, 0)),
          pl.BlockSpec((D, D), lambda i: (0, 0)),
          pl.BlockSpec((D, D), lambda i: (0, 0)),
      ],
      out_specs=pl.BlockSpec((BLK, D), lambda i: (i, 0)),
      out_shape=jax.ShapeDtypeStruct((N_NODES, D), jnp.float32),
  )(agg, h, w_a, w_h, w_pool)


def kernel(x, edge_index, W_input, W_layer0, W_layer1, W_pool):
  src = edge_index[0].astype(jnp.int32).reshape(NW, NCHUNK, CH)
  dst = edge_index[1].astype(jnp.int32).reshape(NW, NCHUNK, CH)
  zeros = jnp.zeros((CH, D), jnp.float32)

  h = _tc_input(x, W_input)
  agg = _segment_sum_sc(h, src, dst, zeros)
  h = _tc_layer(agg, h, W_layer0[:D], W_layer0[D:])
  agg = _segment_sum_sc(h, src, dst, zeros)
  return _tc_layer_pool(agg, h, W_layer1[:D], W_layer1[D:], W_pool)


# speedup vs baseline: 9.5845x; 9.5845x over previous
import functools

import jax
import jax.numpy as jnp
import numpy as np
from jax import lax
from jax.experimental import pallas as pl
from jax.experimental.pallas import tpu as pltpu
from jax.experimental.pallas import tpu_sc as plsc

N_NODES = 10000
N_EDGES = 320000
D = 128
DH = D // 2
BOUND = 5.0

NC = 2
NS = 16
CH = 128
NCHUNK = 158
E_PAD = NS * NCHUNK * CH
N_PAD = 10240
ROWS_PER_TILE = N_PAD // NS
ZCH = ROWS_PER_TILE // CH

_PAD_E = E_PAD - N_EDGES
_PAD_SRC = np.arange(_PAD_E, dtype=np.int32) % N_NODES
_PAD_DST = N_NODES + np.arange(_PAD_E, dtype=np.int32) % (N_PAD - N_NODES)


def _segment_sum_sc(x_split, src, dst, zeros):
  mesh = plsc.VectorSubcoreMesh(core_axis_name="c", subcore_axis_name="s")

  @functools.partial(
      pl.kernel,
      out_type=jax.ShapeDtypeStruct((NC, N_PAD, DH), jnp.float32),
      mesh=mesh,
      compiler_params=pltpu.CompilerParams(use_tc_tiling_on_sc=False),
      scratch_types=[
          pltpu.VMEM((NCHUNK, CH), jnp.int32),
          pltpu.VMEM((NCHUNK, CH), jnp.int32),
          pltpu.VMEM((CH, DH), jnp.float32),
          pltpu.VMEM((CH, DH), jnp.float32),
          pltpu.VMEM((CH, DH), jnp.float32),
          pltpu.VMEM_SHARED((N_PAD, DH), jnp.float32),
          pltpu.SemaphoreType.DMA,
          pltpu.SemaphoreType.DMA,
      ],
  )
  def kern(x_hbm, src_hbm, dst_hbm, zeros_hbm, out_hbm,
           src_v, dst_v, buf0, buf1, zbuf, agg_sh, sem0, sem1):
    c = lax.axis_index("c")
    s = lax.axis_index("s")
    wid = c * NS + s
    pltpu.sync_copy(src_hbm.at[wid], src_v)
    pltpu.sync_copy(dst_hbm.at[s], dst_v)
    pltpu.sync_copy(zeros_hbm, zbuf)
    row0 = s * ROWS_PER_TILE
    for r in range(ZCH):
      pltpu.sync_copy(zbuf, agg_sh.at[pl.ds(row0 + r * CH, CH)])
    plsc.subcore_barrier()

    pltpu.async_copy(x_hbm.at[src_v.at[0]], buf0, sem0)
    pltpu.async_copy(x_hbm.at[src_v.at[1]], buf1, sem1)

    def body(g2, carry):
      c0 = g2 * 2
      c1 = c0 + 1
      pltpu.make_async_copy(x_hbm.at[src_v.at[c0]], buf0, sem0).wait()
      pltpu.sync_copy(buf0, agg_sh.at[dst_v.at[c0]], add=True)

      @pl.when(c0 + 2 < NCHUNK)
      def _():
        pltpu.async_copy(x_hbm.at[src_v.at[c0 + 2]], buf0, sem0)

      pltpu.make_async_copy(x_hbm.at[src_v.at[c1]], buf1, sem1).wait()
      pltpu.sync_copy(buf1, agg_sh.at[dst_v.at[c1]], add=True)

      @pl.when(c1 + 2 < NCHUNK)
      def _():
        pltpu.async_copy(x_hbm.at[src_v.at[c1 + 2]], buf1, sem1)

      return carry

    lax.fori_loop(0, NCHUNK // 2, body, 0)
    plsc.subcore_barrier()
    for r in range(ZCH):
      sl = pl.ds(row0 + r * CH, CH)
      pltpu.sync_copy(agg_sh.at[sl], zbuf)
      pltpu.sync_copy(zbuf, out_hbm.at[c].at[sl])

  return kern(x_split, src, dst, zeros)


BLK = 1000
_DENSE_OUT = (
    jax.ShapeDtypeStruct((N_NODES, D), jnp.float32),
    jax.ShapeDtypeStruct((NC, N_NODES, DH), jnp.float32),
)
_DENSE_OUT_SPECS = (
    pl.BlockSpec((BLK, D), lambda i: (i, 0)),
    pl.BlockSpec((NC, BLK, DH), lambda i: (0, i, 0)),
)


def _softmax(z):
  z = z - jnp.max(z, axis=-1, keepdims=True)
  e = jnp.exp(z)
  return e / jnp.sum(e, axis=-1, keepdims=True)


def _write_split(y, o_ref, o2_ref):
  o_ref[...] = y
  o2_ref[0] = y[:, :DH]
  o2_ref[1] = y[:, DH:]


def _tc_input(x, w):
  def body(x_ref, w_ref, o_ref, o2_ref):
    z = jnp.dot(x_ref[...], w_ref[...], preferred_element_type=jnp.float32)
    _write_split(_softmax(z), o_ref, o2_ref)

  return pl.pallas_call(
      body,
      grid=(N_NODES // BLK,),
      in_specs=[
          pl.BlockSpec((BLK, D), lambda i: (i, 0)),
          pl.BlockSpec((D, D), lambda i: (0, 0)),
      ],
      out_specs=_DENSE_OUT_SPECS,
      out_shape=_DENSE_OUT,
  )(x, w)


def _agg_combined(a_ref):
  a = jnp.concatenate((a_ref[0], a_ref[1]), axis=-1)
  return jnp.clip(a, 0.0, BOUND)


def _tc_layer(agg, h, w_a, w_h):
  def body(a_ref, h_ref, wa_ref, wh_ref, o_ref, o2_ref):
    a = _agg_combined(a_ref)
    z = jnp.dot(a, wa_ref[...], preferred_element_type=jnp.float32)
    z = z + jnp.dot(h_ref[...], wh_ref[...], preferred_element_type=jnp.float32)
    _write_split(_softmax(z), o_ref, o2_ref)

  return pl.pallas_call(
      body,
      grid=(N_NODES // BLK,),
      in_specs=[
          pl.BlockSpec((NC, BLK, DH), lambda i: (0, i, 0)),
          pl.BlockSpec((BLK, D), lambda i: (i, 0)),
          pl.BlockSpec((D, D), lambda i: (0, 0)),
          pl.BlockSpec((D, D), lambda i: (0, 0)),
      ],
      out_specs=_DENSE_OUT_SPECS,
      out_shape=_DENSE_OUT,
  )(agg, h, w_a, w_h)


def _tc_layer_pool(agg, h, w_a, w_h, w_pool):
  def body(a_ref, h_ref, wa_ref, wh_ref, wp_ref, o_ref):
    a = _agg_combined(a_ref)
    z = jnp.dot(a, wa_ref[...], preferred_element_type=jnp.float32)
    z = z + jnp.dot(h_ref[...], wh_ref[...], preferred_element_type=jnp.float32)
    h1 = _softmax(z)
    o_ref[...] = _softmax(
        jnp.dot(h1, wp_ref[...], preferred_element_type=jnp.float32))

  return pl.pallas_call(
      body,
      grid=(N_NODES // BLK,),
      in_specs=[
          pl.BlockSpec((NC, BLK, DH), lambda i: (0, i, 0)),
          pl.BlockSpec((BLK, D), lambda i: (i, 0)),
          pl.BlockSpec((D, D), lambda i: (0, 0)),
          pl.BlockSpec((D, D), lambda i: (0, 0)),
          pl.BlockSpec((D, D), lambda i: (0, 0)),
      ],
      out_specs=pl.BlockSpec((BLK, D), lambda i: (i, 0)),
      out_shape=jax.ShapeDtypeStruct((N_NODES, D), jnp.float32),
  )(agg, h, w_a, w_h, w_pool)


def kernel(x, edge_index, W_input, W_layer0, W_layer1, W_pool):
  src = jnp.concatenate(
      (edge_index[0].astype(jnp.int32), jnp.asarray(_PAD_SRC)))
  dst = jnp.concatenate(
      (edge_index[1].astype(jnp.int32), jnp.asarray(_PAD_DST)))
  src = src.reshape(NS, NCHUNK, CH)
  src = jnp.stack((src, src + N_NODES)).reshape(NC * NS, NCHUNK, CH)
  dst = dst.reshape(NS, NCHUNK, CH)
  zeros = jnp.zeros((CH, DH), jnp.float32)

  h, hs = _tc_input(x, W_input)
  agg = _segment_sum_sc(hs.reshape(NC * N_NODES, DH), src, dst, zeros)
  h, hs = _tc_layer(agg, h, W_layer0[:D], W_layer0[D:])
  agg = _segment_sum_sc(hs.reshape(NC * N_NODES, DH), src, dst, zeros)
  return _tc_layer_pool(agg, h, W_layer1[:D], W_layer1[D:], W_pool)


# speedup vs baseline: 10.8337x; 1.1303x over previous
import functools

import jax
import jax.numpy as jnp
import numpy as np
from jax import lax
from jax.experimental import pallas as pl
from jax.experimental.pallas import tpu as pltpu
from jax.experimental.pallas import tpu_sc as plsc

N_NODES = 10000
N_EDGES = 320000
D = 128
DH = D // 2
BOUND = 5.0

NC = 2
NS = 16
CH = 128
NCHUNK = 160
E_PAD = NS * NCHUNK * CH
N_PAD = 10240
ROWS_PER_TILE = N_PAD // NS
ZCH = ROWS_PER_TILE // CH

_PAD_E = E_PAD - N_EDGES
_PAD_SRC = np.arange(_PAD_E, dtype=np.int32) % N_NODES
_PAD_DST = N_NODES + np.arange(_PAD_E, dtype=np.int32) % (N_PAD - N_NODES)


def _segment_sum_sc(x_split, src, dst, zeros):
  mesh = plsc.VectorSubcoreMesh(core_axis_name="c", subcore_axis_name="s")

  @functools.partial(
      pl.kernel,
      out_type=jax.ShapeDtypeStruct((NC, N_PAD, DH), jnp.float32),
      mesh=mesh,
      compiler_params=pltpu.CompilerParams(use_tc_tiling_on_sc=False),
      scratch_types=[
          pltpu.VMEM((NCHUNK, CH), jnp.int32),
          pltpu.VMEM((NCHUNK, CH), jnp.int32),
          [pltpu.VMEM((CH, DH), jnp.float32)] * 4,
          pltpu.VMEM((CH, DH), jnp.float32),
          pltpu.VMEM_SHARED((N_PAD, DH), jnp.float32),
          [pltpu.SemaphoreType.DMA] * 4,
          [pltpu.SemaphoreType.DMA] * 4,
      ],
  )
  def kern(x_hbm, src_hbm, dst_hbm, zeros_hbm, out_hbm,
           src_v, dst_v, bufs, zbuf, agg_sh, sem_g, sem_s):
    c = lax.axis_index("c")
    s = lax.axis_index("s")
    wid = c * NS + s
    pltpu.sync_copy(src_hbm.at[wid], src_v)
    pltpu.sync_copy(dst_hbm.at[s], dst_v)
    pltpu.sync_copy(zeros_hbm, zbuf)
    row0 = s * ROWS_PER_TILE
    for r in range(ZCH):
      pltpu.sync_copy(zbuf, agg_sh.at[pl.ds(row0 + r * CH, CH)])
    plsc.subcore_barrier()

    def fire_gather(slot, chunk):
      pltpu.async_copy(x_hbm.at[src_v.at[chunk]], bufs[slot], sem_g[slot])

    def wait_gather(slot, chunk):
      pltpu.make_async_copy(
          x_hbm.at[src_v.at[chunk]], bufs[slot], sem_g[slot]).wait()

    def fire_scatter(slot, chunk):
      pltpu.async_copy(
          bufs[slot], agg_sh.at[dst_v.at[chunk]], sem_s[slot], add=True)

    def wait_scatter(slot):
      pltpu.make_async_copy(
          bufs[slot], agg_sh.at[dst_v.at[0]], sem_s[slot]).wait()

    fire_gather(0, 0)
    fire_gather(1, 1)

    def body(g4, carry):
      for j in range(4):
        ck = g4 * 4 + j
        cf = ck + 2
        slot_f = (j + 2) % 4

        @pl.when(cf < NCHUNK)
        def _():
          @pl.when(cf >= 4)
          def _():
            wait_scatter(slot_f)
          fire_gather(slot_f, cf)

        wait_gather(j, ck)
        fire_scatter(j, ck)
      return carry

    lax.fori_loop(0, NCHUNK // 4, body, 0)
    for b in range(4):
      wait_scatter(b)
    plsc.subcore_barrier()
    for r in range(ZCH):
      sl = pl.ds(row0 + r * CH, CH)
      pltpu.sync_copy(agg_sh.at[sl], zbuf)
      pltpu.sync_copy(zbuf, out_hbm.at[c].at[sl])

  return kern(x_split, src, dst, zeros)


BLK = 1000
_DENSE_OUT = (
    jax.ShapeDtypeStruct((N_NODES, D), jnp.float32),
    jax.ShapeDtypeStruct((NC, N_NODES, DH), jnp.float32),
)
_DENSE_OUT_SPECS = (
    pl.BlockSpec((BLK, D), lambda i: (i, 0)),
    pl.BlockSpec((NC, BLK, DH), lambda i: (0, i, 0)),
)


def _softmax(z):
  z = z - jnp.max(z, axis=-1, keepdims=True)
  e = jnp.exp(z)
  return e / jnp.sum(e, axis=-1, keepdims=True)


def _write_split(y, o_ref, o2_ref):
  o_ref[...] = y
  o2_ref[0] = y[:, :DH]
  o2_ref[1] = y[:, DH:]


def _tc_input(x, w):
  def body(x_ref, w_ref, o_ref, o2_ref):
    z = jnp.dot(x_ref[...], w_ref[...], preferred_element_type=jnp.float32)
    _write_split(_softmax(z), o_ref, o2_ref)

  return pl.pallas_call(
      body,
      grid=(N_NODES // BLK,),
      in_specs=[
          pl.BlockSpec((BLK, D), lambda i: (i, 0)),
          pl.BlockSpec((D, D), lambda i: (0, 0)),
      ],
      out_specs=_DENSE_OUT_SPECS,
      out_shape=_DENSE_OUT,
  )(x, w)


def _agg_combined(a_ref):
  a = jnp.concatenate((a_ref[0], a_ref[1]), axis=-1)
  return jnp.clip(a, 0.0, BOUND)


def _tc_layer(agg, h, w_a, w_h):
  def body(a_ref, h_ref, wa_ref, wh_ref, o_ref, o2_ref):
    a = _agg_combined(a_ref)
    z = jnp.dot(a, wa_ref[...], preferred_element_type=jnp.float32)
    z = z + jnp.dot(h_ref[...], wh_ref[...], preferred_element_type=jnp.float32)
    _write_split(_softmax(z), o_ref, o2_ref)

  return pl.pallas_call(
      body,
      grid=(N_NODES // BLK,),
      in_specs=[
          pl.BlockSpec((NC, BLK, DH), lambda i: (0, i, 0)),
          pl.BlockSpec((BLK, D), lambda i: (i, 0)),
          pl.BlockSpec((D, D), lambda i: (0, 0)),
          pl.BlockSpec((D, D), lambda i: (0, 0)),
      ],
      out_specs=_DENSE_OUT_SPECS,
      out_shape=_DENSE_OUT,
  )(agg, h, w_a, w_h)


def _tc_layer_pool(agg, h, w_a, w_h, w_pool):
  def body(a_ref, h_ref, wa_ref, wh_ref, wp_ref, o_ref):
    a = _agg_combined(a_ref)
    z = jnp.dot(a, wa_ref[...], preferred_element_type=jnp.float32)
    z = z + jnp.dot(h_ref[...], wh_ref[...], preferred_element_type=jnp.float32)
    h1 = _softmax(z)
    o_ref[...] = _softmax(
        jnp.dot(h1, wp_ref[...], preferred_element_type=jnp.float32))

  return pl.pallas_call(
      body,
      grid=(N_NODES // BLK,),
      in_specs=[
          pl.BlockSpec((NC, BLK, DH), lambda i: (0, i, 0)),
          pl.BlockSpec((BLK, D), lambda i: (i, 0)),
          pl.BlockSpec((D, D), lambda i: (0, 0)),
          pl.BlockSpec((D, D), lambda i: (0, 0)),
          pl.BlockSpec((D, D), lambda i: (0, 0)),
      ],
      out_specs=pl.BlockSpec((BLK, D), lambda i: (i, 0)),
      out_shape=jax.ShapeDtypeStruct((N_NODES, D), jnp.float32),
  )(agg, h, w_a, w_h, w_pool)


def kernel(x, edge_index, W_input, W_layer0, W_layer1, W_pool):
  src = jnp.concatenate(
      (edge_index[0].astype(jnp.int32), jnp.asarray(_PAD_SRC)))
  dst = jnp.concatenate(
      (edge_index[1].astype(jnp.int32), jnp.asarray(_PAD_DST)))
  src = src.reshape(NS, NCHUNK, CH)
  src = jnp.stack((src, src + N_NODES)).reshape(NC * NS, NCHUNK, CH)
  dst = dst.reshape(NS, NCHUNK, CH)
  zeros = jnp.zeros((CH, DH), jnp.float32)

  h, hs = _tc_input(x, W_input)
  agg = _segment_sum_sc(hs.reshape(NC * N_NODES, DH), src, dst, zeros)
  h, hs = _tc_layer(agg, h, W_layer0[:D], W_layer0[D:])
  agg = _segment_sum_sc(hs.reshape(NC * N_NODES, DH), src, dst, zeros)
  return _tc_layer_pool(agg, h, W_layer1[:D], W_layer1[D:], W_pool)


# speedup vs baseline: 12.8901x; 1.1898x over previous
import functools

import jax
import jax.numpy as jnp
import numpy as np
from jax import lax
from jax.experimental import pallas as pl
from jax.experimental.pallas import tpu as pltpu
from jax.experimental.pallas import tpu_sc as plsc

N_NODES = 10000
N_EDGES = 320000
D = 128
DH = D // 2
BOUND = 5.0

NC = 2
NS = 16
CH = 128
NCHUNK = 160
E_PAD = NS * NCHUNK * CH
N_PAD = 10240
ROWS_PER_TILE = N_PAD // NS
ZCH = ROWS_PER_TILE // CH

_PAD_E = E_PAD - N_EDGES
_PAD_SRC = np.arange(_PAD_E, dtype=np.int32) % N_NODES
_PAD_DST = N_NODES + np.arange(_PAD_E, dtype=np.int32) % (N_PAD - N_NODES)

_OUT_IDX = (2 * np.arange(N_PAD, dtype=np.int32)[None, :]
            + np.arange(NC, dtype=np.int32)[:, None]).reshape(
                NC * NS, ZCH, CH)


def _segment_sum_sc(x2, src, dst, zeros, out_idx):
  mesh = plsc.VectorSubcoreMesh(core_axis_name="c", subcore_axis_name="s")

  @functools.partial(
      pl.kernel,
      out_type=jax.ShapeDtypeStruct((NC * N_PAD, DH), jnp.float32),
      mesh=mesh,
      compiler_params=pltpu.CompilerParams(use_tc_tiling_on_sc=False),
      scratch_types=[
          pltpu.VMEM((NCHUNK, CH), jnp.int32),
          pltpu.VMEM((NCHUNK, CH), jnp.int32),
          pltpu.VMEM((ZCH, CH), jnp.int32),
          [pltpu.VMEM((CH, DH), jnp.float32)] * 4,
          pltpu.VMEM((CH, DH), jnp.float32),
          pltpu.VMEM_SHARED((N_PAD, DH), jnp.float32),
          [pltpu.SemaphoreType.DMA] * 4,
          [pltpu.SemaphoreType.DMA] * 4,
      ],
  )
  def kern(x_hbm, src_hbm, dst_hbm, zeros_hbm, oidx_hbm, out_hbm,
           src_v, dst_v, oidx_v, bufs, zbuf, agg_sh, sem_g, sem_s):
    c = lax.axis_index("c")
    s = lax.axis_index("s")
    wid = c * NS + s
    pltpu.sync_copy(src_hbm.at[wid], src_v)
    pltpu.sync_copy(dst_hbm.at[s], dst_v)
    pltpu.sync_copy(oidx_hbm.at[wid], oidx_v)
    pltpu.sync_copy(zeros_hbm, zbuf)
    row0 = s * ROWS_PER_TILE
    for r in range(ZCH):
      pltpu.sync_copy(zbuf, agg_sh.at[pl.ds(row0 + r * CH, CH)])
    plsc.subcore_barrier()

    def fire_gather(slot, chunk):
      pltpu.async_copy(x_hbm.at[src_v.at[chunk]], bufs[slot], sem_g[slot])

    def wait_gather(slot, chunk):
      pltpu.make_async_copy(
          x_hbm.at[src_v.at[chunk]], bufs[slot], sem_g[slot]).wait()

    def fire_scatter(slot, chunk):
      pltpu.async_copy(
          bufs[slot], agg_sh.at[dst_v.at[chunk]], sem_s[slot], add=True)

    def wait_scatter(slot):
      pltpu.make_async_copy(
          bufs[slot], agg_sh.at[dst_v.at[0]], sem_s[slot]).wait()

    fire_gather(0, 0)
    fire_gather(1, 1)

    def body(g4, carry):
      for j in range(4):
        ck = g4 * 4 + j
        cf = ck + 2
        slot_f = (j + 2) % 4

        @pl.when(cf < NCHUNK)
        def _():
          @pl.when(cf >= 4)
          def _():
            wait_scatter(slot_f)
          fire_gather(slot_f, cf)

        wait_gather(j, ck)
        fire_scatter(j, ck)
      return carry

    lax.fori_loop(0, NCHUNK // 4, body, 0)
    for b in range(4):
      wait_scatter(b)
    plsc.subcore_barrier()
    for r in range(ZCH):
      pltpu.sync_copy(agg_sh.at[pl.ds(row0 + r * CH, CH)], zbuf)
      pltpu.sync_copy(zbuf, out_hbm.at[oidx_v.at[r]])

  return kern(x2, src, dst, zeros, out_idx)


BLK = 1000


def _softmax(z):
  z = z - jnp.max(z, axis=-1, keepdims=True)
  e = jnp.exp(z)
  return e / jnp.sum(e, axis=-1, keepdims=True)


def _tc_input(x, w):
  def body(x_ref, w_ref, o_ref):
    z = jnp.dot(x_ref[...], w_ref[...], preferred_element_type=jnp.float32)
    o_ref[...] = _softmax(z)

  return pl.pallas_call(
      body,
      grid=(N_NODES // BLK,),
      in_specs=[
          pl.BlockSpec((BLK, D), lambda i: (i, 0)),
          pl.BlockSpec((D, D), lambda i: (0, 0)),
      ],
      out_specs=pl.BlockSpec((BLK, D), lambda i: (i, 0)),
      out_shape=jax.ShapeDtypeStruct((N_NODES, D), jnp.float32),
  )(x, w)


def _tc_layer(agg, h, w_a, w_h):
  def body(a_ref, h_ref, wa_ref, wh_ref, o_ref):
    a = jnp.clip(a_ref[...], 0.0, BOUND)
    z = jnp.dot(a, wa_ref[...], preferred_element_type=jnp.float32)
    z = z + jnp.dot(h_ref[...], wh_ref[...], preferred_element_type=jnp.float32)
    o_ref[...] = _softmax(z)

  return pl.pallas_call(
      body,
      grid=(N_NODES // BLK,),
      in_specs=[
          pl.BlockSpec((BLK, D), lambda i: (i, 0)),
          pl.BlockSpec((BLK, D), lambda i: (i, 0)),
          pl.BlockSpec((D, D), lambda i: (0, 0)),
          pl.BlockSpec((D, D), lambda i: (0, 0)),
      ],
      out_specs=pl.BlockSpec((BLK, D), lambda i: (i, 0)),
      out_shape=jax.ShapeDtypeStruct((N_NODES, D), jnp.float32),
  )(agg, h, w_a, w_h)


def _tc_layer_pool(agg, h, w_a, w_h, w_pool):
  def body(a_ref, h_ref, wa_ref, wh_ref, wp_ref, o_ref):
    a = jnp.clip(a_ref[...], 0.0, BOUND)
    z = jnp.dot(a, wa_ref[...], preferred_element_type=jnp.float32)
    z = z + jnp.dot(h_ref[...], wh_ref[...], preferred_element_type=jnp.float32)
    h1 = _softmax(z)
    o_ref[...] = _softmax(
        jnp.dot(h1, wp_ref[...], preferred_element_type=jnp.float32))

  return pl.pallas_call(
      body,
      grid=(N_NODES // BLK,),
      in_specs=[
          pl.BlockSpec((BLK, D), lambda i: (i, 0)),
          pl.BlockSpec((BLK, D), lambda i: (i, 0)),
          pl.BlockSpec((D, D), lambda i: (0, 0)),
          pl.BlockSpec((D, D), lambda i: (0, 0)),
          pl.BlockSpec((D, D), lambda i: (0, 0)),
      ],
      out_specs=pl.BlockSpec((BLK, D), lambda i: (i, 0)),
      out_shape=jax.ShapeDtypeStruct((N_NODES, D), jnp.float32),
  )(agg, h, w_a, w_h, w_pool)


def kernel(x, edge_index, W_input, W_layer0, W_layer1, W_pool):
  src = jnp.concatenate(
      (edge_index[0].astype(jnp.int32), jnp.asarray(_PAD_SRC)))
  dst = jnp.concatenate(
      (edge_index[1].astype(jnp.int32), jnp.asarray(_PAD_DST)))
  src2 = 2 * src.reshape(NS, NCHUNK, CH)
  src2 = jnp.stack((src2, src2 + 1)).reshape(NC * NS, NCHUNK, CH)
  dst = dst.reshape(NS, NCHUNK, CH)
  zeros = jnp.zeros((CH, DH), jnp.float32)
  out_idx = jnp.asarray(_OUT_IDX)

  h = _tc_input(x, W_input)
  agg = _segment_sum_sc(
      h.reshape(NC * N_NODES, DH), src2, dst, zeros, out_idx)
  h = _tc_layer(agg.reshape(N_PAD, D), h, W_layer0[:D], W_layer0[D:])
  agg = _segment_sum_sc(
      h.reshape(NC * N_NODES, DH), src2, dst, zeros, out_idx)
  return _tc_layer_pool(
      agg.reshape(N_PAD, D), h, W_layer1[:D], W_layer1[D:], W_pool)


# speedup vs baseline: 13.8793x; 1.0767x over previous
import functools

import jax
import jax.numpy as jnp
import numpy as np
from jax import lax
from jax.experimental import pallas as pl
from jax.experimental.pallas import tpu as pltpu
from jax.experimental.pallas import tpu_sc as plsc

N_NODES = 10000
N_EDGES = 320000
D = 128
DH = D // 2
BOUND = 5.0

NC = 2
NS = 16
CH = 128
NCHUNK = 160
E_PAD = NS * NCHUNK * CH
N_PAD = 10240
ROWS_PER_TILE = N_PAD // NS
ZCH = ROWS_PER_TILE // CH

_PAD_E = E_PAD - N_EDGES
_PAD_ROWS = _PAD_E // CH
_E_ROWS = N_EDGES // CH

_OUT_IDX = (2 * np.arange(N_PAD, dtype=np.int32)[None, :]
            + np.arange(NC, dtype=np.int32)[:, None]).reshape(
                NC * NS, ZCH, CH)


def _prep_indices(edge_index):
  def body(ei_ref, src2_ref, dst_ref):
    e = ei_ref[...]
    s2 = 2 * e[0].reshape(_E_ROWS, CH)
    d2 = e[1].reshape(_E_ROWS, CH)
    f = (jax.lax.broadcasted_iota(jnp.int32, (_PAD_ROWS, CH), 0) * CH
         + jax.lax.broadcasted_iota(jnp.int32, (_PAD_ROWS, CH), 1))
    ps2 = 2 * (f % N_NODES)
    pd2 = N_NODES + f % (N_PAD - N_NODES)
    src2_ref[0, : _E_ROWS] = s2
    src2_ref[0, _E_ROWS:] = ps2
    src2_ref[1, : _E_ROWS] = s2 + 1
    src2_ref[1, _E_ROWS:] = ps2 + 1
    dst_ref[: _E_ROWS] = d2
    dst_ref[_E_ROWS:] = pd2

  return pl.pallas_call(
      body,
      out_shape=(
          jax.ShapeDtypeStruct((NC, NS * NCHUNK, CH), jnp.int32),
          jax.ShapeDtypeStruct((NS * NCHUNK, CH), jnp.int32),
      ),
  )(edge_index)


def _segment_sum_sc(x2, src, dst, zeros, out_idx):
  mesh = plsc.VectorSubcoreMesh(core_axis_name="c", subcore_axis_name="s")

  @functools.partial(
      pl.kernel,
      out_type=jax.ShapeDtypeStruct((NC * N_PAD, DH), jnp.float32),
      mesh=mesh,
      compiler_params=pltpu.CompilerParams(use_tc_tiling_on_sc=False),
      scratch_types=[
          pltpu.VMEM((NCHUNK, CH), jnp.int32),
          pltpu.VMEM((NCHUNK, CH), jnp.int32),
          pltpu.VMEM((ZCH, CH), jnp.int32),
          [pltpu.VMEM((CH, DH), jnp.float32)] * 4,
          pltpu.VMEM((CH, DH), jnp.float32),
          pltpu.VMEM_SHARED((N_PAD, DH), jnp.float32),
          [pltpu.SemaphoreType.DMA] * 4,
          [pltpu.SemaphoreType.DMA] * 4,
      ],
  )
  def kern(x_hbm, src_hbm, dst_hbm, zeros_hbm, oidx_hbm, out_hbm,
           src_v, dst_v, oidx_v, bufs, zbuf, agg_sh, sem_g, sem_s):
    c = lax.axis_index("c")
    s = lax.axis_index("s")
    wid = c * NS + s
    pltpu.sync_copy(src_hbm.at[wid], src_v)
    pltpu.sync_copy(dst_hbm.at[s], dst_v)
    pltpu.sync_copy(oidx_hbm.at[wid], oidx_v)
    pltpu.sync_copy(zeros_hbm, zbuf)
    row0 = s * ROWS_PER_TILE
    for r in range(ZCH):
      pltpu.sync_copy(zbuf, agg_sh.at[pl.ds(row0 + r * CH, CH)])
    plsc.subcore_barrier()

    def fire_gather(slot, chunk):
      pltpu.async_copy(x_hbm.at[src_v.at[chunk]], bufs[slot], sem_g[slot])

    def wait_gather(slot, chunk):
      pltpu.make_async_copy(
          x_hbm.at[src_v.at[chunk]], bufs[slot], sem_g[slot]).wait()

    def fire_scatter(slot, chunk):
      pltpu.async_copy(
          bufs[slot], agg_sh.at[dst_v.at[chunk]], sem_s[slot], add=True)

    def wait_scatter(slot):
      pltpu.make_async_copy(
          bufs[slot], agg_sh.at[dst_v.at[0]], sem_s[slot]).wait()

    fire_gather(0, 0)
    fire_gather(1, 1)

    def body(g4, carry):
      for j in range(4):
        ck = g4 * 4 + j
        cf = ck + 2
        slot_f = (j + 2) % 4

        @pl.when(cf < NCHUNK)
        def _():
          @pl.when(cf >= 4)
          def _():
            wait_scatter(slot_f)
          fire_gather(slot_f, cf)

        wait_gather(j, ck)
        fire_scatter(j, ck)
      return carry

    lax.fori_loop(0, NCHUNK // 4, body, 0)
    for b in range(4):
      wait_scatter(b)
    plsc.subcore_barrier()
    for r in range(ZCH):
      pltpu.sync_copy(agg_sh.at[pl.ds(row0 + r * CH, CH)], zbuf)
      pltpu.sync_copy(zbuf, out_hbm.at[oidx_v.at[r]])

  return kern(x2, src, dst, zeros, out_idx)


BLK = 2000


def _softmax(z):
  z = z - jnp.max(z, axis=-1, keepdims=True)
  e = jnp.exp(z)
  return e / jnp.sum(e, axis=-1, keepdims=True)


def _tc_input(x, w):
  def body(x_ref, w_ref, o_ref):
    z = jnp.dot(x_ref[...], w_ref[...], preferred_element_type=jnp.float32)
    o_ref[...] = _softmax(z)

  return pl.pallas_call(
      body,
      grid=(N_NODES // BLK,),
      in_specs=[
          pl.BlockSpec((BLK, D), lambda i: (i, 0)),
          pl.BlockSpec((D, D), lambda i: (0, 0)),
      ],
      out_specs=pl.BlockSpec((BLK, D), lambda i: (i, 0)),
      out_shape=jax.ShapeDtypeStruct((N_NODES, D), jnp.float32),
  )(x, w)


def _tc_layer(agg, h, w_a, w_h):
  def body(a_ref, h_ref, wa_ref, wh_ref, o_ref):
    a = jnp.clip(a_ref[...], 0.0, BOUND)
    z = jnp.dot(a, wa_ref[...], preferred_element_type=jnp.float32)
    z = z + jnp.dot(h_ref[...], wh_ref[...], preferred_element_type=jnp.float32)
    o_ref[...] = _softmax(z)

  return pl.pallas_call(
      body,
      grid=(N_NODES // BLK,),
      in_specs=[
          pl.BlockSpec((BLK, D), lambda i: (i, 0)),
          pl.BlockSpec((BLK, D), lambda i: (i, 0)),
          pl.BlockSpec((D, D), lambda i: (0, 0)),
          pl.BlockSpec((D, D), lambda i: (0, 0)),
      ],
      out_specs=pl.BlockSpec((BLK, D), lambda i: (i, 0)),
      out_shape=jax.ShapeDtypeStruct((N_NODES, D), jnp.float32),
  )(agg, h, w_a, w_h)


def _tc_layer_pool(agg, h, w_a, w_h, w_pool):
  def body(a_ref, h_ref, wa_ref, wh_ref, wp_ref, o_ref):
    a = jnp.clip(a_ref[...], 0.0, BOUND)
    z = jnp.dot(a, wa_ref[...], preferred_element_type=jnp.float32)
    z = z + jnp.dot(h_ref[...], wh_ref[...], preferred_element_type=jnp.float32)
    h1 = _softmax(z)
    o_ref[...] = _softmax(
        jnp.dot(h1, wp_ref[...], preferred_element_type=jnp.float32))

  return pl.pallas_call(
      body,
      grid=(N_NODES // BLK,),
      in_specs=[
          pl.BlockSpec((BLK, D), lambda i: (i, 0)),
          pl.BlockSpec((BLK, D), lambda i: (i, 0)),
          pl.BlockSpec((D, D), lambda i: (0, 0)),
          pl.BlockSpec((D, D), lambda i: (0, 0)),
          pl.BlockSpec((D, D), lambda i: (0, 0)),
      ],
      out_specs=pl.BlockSpec((BLK, D), lambda i: (i, 0)),
      out_shape=jax.ShapeDtypeStruct((N_NODES, D), jnp.float32),
  )(agg, h, w_a, w_h, w_pool)


def kernel(x, edge_index, W_input, W_layer0, W_layer1, W_pool):
  src2, dst = _prep_indices(edge_index.astype(jnp.int32))
  src2 = src2.reshape(NC * NS, NCHUNK, CH)
  dst = dst.reshape(NS, NCHUNK, CH)
  zeros = jnp.zeros((CH, DH), jnp.float32)
  out_idx = jnp.asarray(_OUT_IDX)

  h = _tc_input(x, W_input)
  agg = _segment_sum_sc(
      h.reshape(NC * N_NODES, DH), src2, dst, zeros, out_idx)
  h = _tc_layer(agg.reshape(N_PAD, D), h, W_layer0[:D], W_layer0[D:])
  agg = _segment_sum_sc(
      h.reshape(NC * N_NODES, DH), src2, dst, zeros, out_idx)
  return _tc_layer_pool(
      agg.reshape(N_PAD, D), h, W_layer1[:D], W_layer1[D:], W_pool)


# speedup vs baseline: 14.0957x; 1.0156x over previous
import functools

import jax
import jax.numpy as jnp
import numpy as np
from jax import lax
from jax.experimental import pallas as pl
from jax.experimental.pallas import tpu as pltpu
from jax.experimental.pallas import tpu_sc as plsc

N_NODES = 10000
N_EDGES = 320000
D = 128
DH = D // 2
BOUND = 5.0

NC = 2
NS = 16
CH = 128
NCHUNK = 160
E_PAD = NS * NCHUNK * CH
N_PAD = 10240
ROWS_PER_TILE = N_PAD // NS
ZCH = ROWS_PER_TILE // CH

_PAD_E = E_PAD - N_EDGES
_PAD_ROWS = _PAD_E // CH
_E_ROWS = N_EDGES // CH

_OUT_IDX = (2 * np.arange(N_PAD, dtype=np.int32)[None, :]
            + np.arange(NC, dtype=np.int32)[:, None]).reshape(
                NC * NS, ZCH, CH)


def _prep_indices(edge_index):
  def body(ei_ref, src2_ref, dst_ref):
    e = ei_ref[...]
    s2 = 2 * e[0].reshape(_E_ROWS, CH)
    d2 = e[1].reshape(_E_ROWS, CH)
    f = (jax.lax.broadcasted_iota(jnp.int32, (_PAD_ROWS, CH), 0) * CH
         + jax.lax.broadcasted_iota(jnp.int32, (_PAD_ROWS, CH), 1))
    ps2 = 2 * (f % N_NODES)
    pd2 = N_NODES + f % (N_PAD - N_NODES)
    src2_ref[0, : _E_ROWS] = s2
    src2_ref[0, _E_ROWS:] = ps2
    src2_ref[1, : _E_ROWS] = s2 + 1
    src2_ref[1, _E_ROWS:] = ps2 + 1
    dst_ref[: _E_ROWS] = d2
    dst_ref[_E_ROWS:] = pd2

  return pl.pallas_call(
      body,
      out_shape=(
          jax.ShapeDtypeStruct((NC, NS * NCHUNK, CH), jnp.int32),
          jax.ShapeDtypeStruct((NS * NCHUNK, CH), jnp.int32),
      ),
  )(edge_index)


def _segment_sum_sc(x2, src, dst, zeros, out_idx):
  mesh = plsc.VectorSubcoreMesh(core_axis_name="c", subcore_axis_name="s")

  @functools.partial(
      pl.kernel,
      out_type=jax.ShapeDtypeStruct((NC * N_PAD, DH), jnp.float32),
      mesh=mesh,
      compiler_params=pltpu.CompilerParams(use_tc_tiling_on_sc=False),
      scratch_types=[
          pltpu.VMEM((NCHUNK, CH), jnp.int32),
          pltpu.VMEM((NCHUNK, CH), jnp.int32),
          pltpu.VMEM((ZCH, CH), jnp.int32),
          [pltpu.VMEM((CH, DH), jnp.float32)] * 4,
          pltpu.VMEM((CH, DH), jnp.float32),
          pltpu.VMEM_SHARED((N_PAD, DH), jnp.float32),
          [pltpu.SemaphoreType.DMA] * 4,
          [pltpu.SemaphoreType.DMA] * 4,
      ],
  )
  def kern(x_hbm, src_hbm, dst_hbm, zeros_hbm, oidx_hbm, out_hbm,
           src_v, dst_v, oidx_v, bufs, zbuf, agg_sh, sem_g, sem_s):
    c = lax.axis_index("c")
    s = lax.axis_index("s")
    wid = c * NS + s

    def fire_gather(slot, chunk):
      pltpu.async_copy(x_hbm.at[src_v.at[chunk]], bufs[slot], sem_g[slot])

    def wait_gather(slot, chunk):
      pltpu.make_async_copy(
          x_hbm.at[src_v.at[chunk]], bufs[slot], sem_g[slot]).wait()

    def fire_scatter(slot, chunk):
      pltpu.async_copy(
          bufs[slot], agg_sh.at[dst_v.at[chunk]], sem_s[slot], add=True)

    def wait_scatter(slot):
      pltpu.make_async_copy(
          bufs[slot], agg_sh.at[dst_v.at[0]], sem_s[slot]).wait()

    pltpu.sync_copy(src_hbm.at[wid], src_v)
    fire_gather(0, 0)
    fire_gather(1, 1)
    pltpu.sync_copy(dst_hbm.at[s], dst_v)
    pltpu.sync_copy(oidx_hbm.at[wid], oidx_v)
    pltpu.sync_copy(zeros_hbm, zbuf)
    row0 = s * ROWS_PER_TILE
    for r in range(ZCH):
      pltpu.sync_copy(zbuf, agg_sh.at[pl.ds(row0 + r * CH, CH)])
    plsc.subcore_barrier()


    def body(g4, carry):
      for j in range(4):
        ck = g4 * 4 + j
        cf = ck + 2
        slot_f = (j + 2) % 4

        @pl.when(cf < NCHUNK)
        def _():
          @pl.when(cf >= 4)
          def _():
            wait_scatter(slot_f)
          fire_gather(slot_f, cf)

        wait_gather(j, ck)
        fire_scatter(j, ck)
      return carry

    lax.fori_loop(0, NCHUNK // 4, body, 0)
    for b in range(4):
      wait_scatter(b)
    plsc.subcore_barrier()
    bufs5 = list(bufs) + [zbuf]
    sems_rd = [sem_g[0], sem_g[1], sem_g[2], sem_g[3], sem_s[0]]
    sems_wr = [sem_s[1], sem_s[2], sem_s[3], sem_g[0], sem_g[1]]
    for r in range(ZCH):
      pltpu.async_copy(
          agg_sh.at[pl.ds(row0 + r * CH, CH)], bufs5[r], sems_rd[r])
    for r in range(ZCH):
      pltpu.make_async_copy(
          agg_sh.at[pl.ds(row0 + r * CH, CH)], bufs5[r], sems_rd[r]).wait()
      pltpu.async_copy(bufs5[r], out_hbm.at[oidx_v.at[r]], sems_wr[r])
    for r in range(ZCH):
      pltpu.make_async_copy(
          bufs5[r], out_hbm.at[oidx_v.at[r]], sems_wr[r]).wait()

  return kern(x2, src, dst, zeros, out_idx)


BLK = 2000


def _softmax(z):
  z = z - jnp.max(z, axis=-1, keepdims=True)
  e = jnp.exp(z)
  return e / jnp.sum(e, axis=-1, keepdims=True)


def _tc_input(x, w):
  def body(x_ref, w_ref, o_ref):
    z = jnp.dot(x_ref[...], w_ref[...], preferred_element_type=jnp.float32)
    o_ref[...] = _softmax(z)

  return pl.pallas_call(
      body,
      grid=(N_NODES // BLK,),
      in_specs=[
          pl.BlockSpec((BLK, D), lambda i: (i, 0)),
          pl.BlockSpec((D, D), lambda i: (0, 0)),
      ],
      out_specs=pl.BlockSpec((BLK, D), lambda i: (i, 0)),
      out_shape=jax.ShapeDtypeStruct((N_NODES, D), jnp.float32),
  )(x, w)


def _tc_layer(agg, h, w_a, w_h):
  def body(a_ref, h_ref, wa_ref, wh_ref, o_ref):
    a = jnp.clip(a_ref[...], 0.0, BOUND)
    z = jnp.dot(a, wa_ref[...], preferred_element_type=jnp.float32)
    z = z + jnp.dot(h_ref[...], wh_ref[...], preferred_element_type=jnp.float32)
    o_ref[...] = _softmax(z)

  return pl.pallas_call(
      body,
      grid=(N_NODES // BLK,),
      in_specs=[
          pl.BlockSpec((BLK, D), lambda i: (i, 0)),
          pl.BlockSpec((BLK, D), lambda i: (i, 0)),
          pl.BlockSpec((D, D), lambda i: (0, 0)),
          pl.BlockSpec((D, D), lambda i: (0, 0)),
      ],
      out_specs=pl.BlockSpec((BLK, D), lambda i: (i, 0)),
      out_shape=jax.ShapeDtypeStruct((N_NODES, D), jnp.float32),
  )(agg, h, w_a, w_h)


def _tc_layer_pool(agg, h, w_a, w_h, w_pool):
  def body(a_ref, h_ref, wa_ref, wh_ref, wp_ref, o_ref):
    a = jnp.clip(a_ref[...], 0.0, BOUND)
    z = jnp.dot(a, wa_ref[...], preferred_element_type=jnp.float32)
    z = z + jnp.dot(h_ref[...], wh_ref[...], preferred_element_type=jnp.float32)
    h1 = _softmax(z)
    o_ref[...] = _softmax(
        jnp.dot(h1, wp_ref[...], preferred_element_type=jnp.float32))

  return pl.pallas_call(
      body,
      grid=(N_NODES // BLK,),
      in_specs=[
          pl.BlockSpec((BLK, D), lambda i: (i, 0)),
          pl.BlockSpec((BLK, D), lambda i: (i, 0)),
          pl.BlockSpec((D, D), lambda i: (0, 0)),
          pl.BlockSpec((D, D), lambda i: (0, 0)),
          pl.BlockSpec((D, D), lambda i: (0, 0)),
      ],
      out_specs=pl.BlockSpec((BLK, D), lambda i: (i, 0)),
      out_shape=jax.ShapeDtypeStruct((N_NODES, D), jnp.float32),
  )(agg, h, w_a, w_h, w_pool)


def kernel(x, edge_index, W_input, W_layer0, W_layer1, W_pool):
  src2, dst = _prep_indices(edge_index.astype(jnp.int32))
  src2 = src2.reshape(NC * NS, NCHUNK, CH)
  dst = dst.reshape(NS, NCHUNK, CH)
  zeros = jnp.zeros((CH, DH), jnp.float32)
  out_idx = jnp.asarray(_OUT_IDX)

  h = _tc_input(x, W_input)
  agg = _segment_sum_sc(
      h.reshape(NC * N_NODES, DH), src2, dst, zeros, out_idx)
  h = _tc_layer(agg.reshape(N_PAD, D), h, W_layer0[:D], W_layer0[D:])
  agg = _segment_sum_sc(
      h.reshape(NC * N_NODES, DH), src2, dst, zeros, out_idx)
  return _tc_layer_pool(
      agg.reshape(N_PAD, D), h, W_layer1[:D], W_layer1[D:], W_pool)


# speedup vs baseline: 14.1274x; 1.0022x over previous
import functools

import jax
import jax.numpy as jnp
import numpy as np
from jax import lax
from jax.experimental import pallas as pl
from jax.experimental.pallas import tpu as pltpu
from jax.experimental.pallas import tpu_sc as plsc

N_NODES = 10000
N_EDGES = 320000
D = 128
DH = D // 2
BOUND = 5.0

NC = 2
NS = 16
CH = 128
NCHUNK = 160
E_PAD = NS * NCHUNK * CH
N_PAD = 10240
ROWS_PER_TILE = N_PAD // NS
ZCH = ROWS_PER_TILE // CH

_PAD_E = E_PAD - N_EDGES
_PAD_ROWS = _PAD_E // CH
_E_ROWS = N_EDGES // CH

_OUT_IDX = (2 * np.arange(N_PAD, dtype=np.int32)[None, :]
            + np.arange(NC, dtype=np.int32)[:, None]).reshape(
                NC * NS, ZCH, CH)


def _prep_indices(edge_index):
  def body(ei_ref, src2_ref, dst_ref):
    e = ei_ref[...]
    s2 = 2 * e[0].reshape(_E_ROWS, CH)
    d2 = e[1].reshape(_E_ROWS, CH)
    f = (jax.lax.broadcasted_iota(jnp.int32, (_PAD_ROWS, CH), 0) * CH
         + jax.lax.broadcasted_iota(jnp.int32, (_PAD_ROWS, CH), 1))
    ps2 = 2 * (f % N_NODES)
    pd2 = N_NODES + f % (N_PAD - N_NODES)
    src2_ref[0, : _E_ROWS] = s2
    src2_ref[0, _E_ROWS:] = ps2
    src2_ref[1, : _E_ROWS] = s2 + 1
    src2_ref[1, _E_ROWS:] = ps2 + 1
    dst_ref[: _E_ROWS] = d2
    dst_ref[_E_ROWS:] = pd2

  return pl.pallas_call(
      body,
      out_shape=(
          jax.ShapeDtypeStruct((NC, NS * NCHUNK, CH), jnp.int32),
          jax.ShapeDtypeStruct((NS * NCHUNK, CH), jnp.int32),
      ),
  )(edge_index)


def _segment_sum_sc(x2, src, dst, zeros, out_idx):
  mesh = plsc.VectorSubcoreMesh(core_axis_name="c", subcore_axis_name="s")

  @functools.partial(
      pl.kernel,
      out_type=jax.ShapeDtypeStruct((NC * N_PAD, DH), jnp.float32),
      mesh=mesh,
      compiler_params=pltpu.CompilerParams(use_tc_tiling_on_sc=False),
      scratch_types=[
          pltpu.VMEM((NCHUNK, CH), jnp.int32),
          pltpu.VMEM((NCHUNK, CH), jnp.int32),
          pltpu.VMEM((ZCH, CH), jnp.int32),
          [pltpu.VMEM((CH, DH), jnp.float32)] * 4,
          pltpu.VMEM((CH, DH), jnp.float32),
          pltpu.VMEM_SHARED((N_PAD, DH), jnp.float32),
          [pltpu.SemaphoreType.DMA] * 4,
          [pltpu.SemaphoreType.DMA] * 4,
      ],
  )
  def kern(x_hbm, src_hbm, dst_hbm, zeros_hbm, oidx_hbm, out_hbm,
           src_v, dst_v, oidx_v, bufs, zbuf, agg_sh, sem_g, sem_s):
    c = lax.axis_index("c")
    s = lax.axis_index("s")
    wid = c * NS + s

    def fire_gather(slot, chunk):
      pltpu.async_copy(x_hbm.at[src_v.at[chunk]], bufs[slot], sem_g[slot])

    def wait_gather(slot, chunk):
      pltpu.make_async_copy(
          x_hbm.at[src_v.at[chunk]], bufs[slot], sem_g[slot]).wait()

    def fire_scatter(slot, chunk):
      pltpu.async_copy(
          bufs[slot], agg_sh.at[dst_v.at[chunk]], sem_s[slot], add=True)

    def wait_scatter(slot):
      pltpu.make_async_copy(
          bufs[slot], agg_sh.at[dst_v.at[0]], sem_s[slot]).wait()

    pltpu.sync_copy(src_hbm.at[wid], src_v)
    fire_gather(0, 0)
    fire_gather(1, 1)
    pltpu.sync_copy(dst_hbm.at[s], dst_v)
    pltpu.sync_copy(oidx_hbm.at[wid], oidx_v)
    pltpu.sync_copy(zeros_hbm, zbuf)
    row0 = s * ROWS_PER_TILE
    for r in range(ZCH):
      pltpu.sync_copy(zbuf, agg_sh.at[pl.ds(row0 + r * CH, CH)])
    plsc.subcore_barrier()


    def body(g4, carry):
      for j in range(4):
        ck = g4 * 4 + j
        cf = ck + 2
        slot_f = (j + 2) % 4

        @pl.when(cf < NCHUNK)
        def _():
          @pl.when(cf >= 4)
          def _():
            wait_scatter(slot_f)
          fire_gather(slot_f, cf)

        wait_gather(j, ck)
        fire_scatter(j, ck)
      return carry

    lax.fori_loop(0, NCHUNK // 4, body, 0)
    for b in range(4):
      wait_scatter(b)
    plsc.subcore_barrier()
    bufs5 = list(bufs) + [zbuf]
    sems_rd = [sem_g[0], sem_g[1], sem_g[2], sem_g[3], sem_s[0]]
    sems_wr = [sem_s[1], sem_s[2], sem_s[3], sem_g[0], sem_g[1]]
    for r in range(ZCH):
      pltpu.async_copy(
          agg_sh.at[pl.ds(row0 + r * CH, CH)], bufs5[r], sems_rd[r])
    for r in range(ZCH):
      pltpu.make_async_copy(
          agg_sh.at[pl.ds(row0 + r * CH, CH)], bufs5[r], sems_rd[r]).wait()
      pltpu.async_copy(bufs5[r], out_hbm.at[oidx_v.at[r]], sems_wr[r])
    for r in range(ZCH):
      pltpu.make_async_copy(
          bufs5[r], out_hbm.at[oidx_v.at[r]], sems_wr[r]).wait()

  return kern(x2, src, dst, zeros, out_idx)


BLK = 2000


def _softmax(z):
  z = z - jnp.max(z, axis=-1, keepdims=True)
  e = jnp.exp(z)
  s = jnp.dot(e, jnp.ones((D, D), jnp.float32),
              preferred_element_type=jnp.float32)
  return e / s


def _tc_input(x, w):
  def body(x_ref, w_ref, o_ref):
    z = jnp.dot(x_ref[...], w_ref[...], preferred_element_type=jnp.float32)
    o_ref[...] = _softmax(z)

  return pl.pallas_call(
      body,
      grid=(N_NODES // BLK,),
      in_specs=[
          pl.BlockSpec((BLK, D), lambda i: (i, 0)),
          pl.BlockSpec((D, D), lambda i: (0, 0)),
      ],
      out_specs=pl.BlockSpec((BLK, D), lambda i: (i, 0)),
      out_shape=jax.ShapeDtypeStruct((N_NODES, D), jnp.float32),
  )(x, w)


def _tc_layer(agg, h, w_a, w_h):
  def body(a_ref, h_ref, wa_ref, wh_ref, o_ref):
    a = jnp.clip(a_ref[...], 0.0, BOUND)
    z = jnp.dot(a, wa_ref[...], preferred_element_type=jnp.float32)
    z = z + jnp.dot(h_ref[...], wh_ref[...], preferred_element_type=jnp.float32)
    o_ref[...] = _softmax(z)

  return pl.pallas_call(
      body,
      grid=(N_NODES // BLK,),
      in_specs=[
          pl.BlockSpec((BLK, D), lambda i: (i, 0)),
          pl.BlockSpec((BLK, D), lambda i: (i, 0)),
          pl.BlockSpec((D, D), lambda i: (0, 0)),
          pl.BlockSpec((D, D), lambda i: (0, 0)),
      ],
      out_specs=pl.BlockSpec((BLK, D), lambda i: (i, 0)),
      out_shape=jax.ShapeDtypeStruct((N_NODES, D), jnp.float32),
  )(agg, h, w_a, w_h)


def _tc_layer_pool(agg, h, w_a, w_h, w_pool):
  def body(a_ref, h_ref, wa_ref, wh_ref, wp_ref, o_ref):
    a = jnp.clip(a_ref[...], 0.0, BOUND)
    z = jnp.dot(a, wa_ref[...], preferred_element_type=jnp.float32)
    z = z + jnp.dot(h_ref[...], wh_ref[...], preferred_element_type=jnp.float32)
    h1 = _softmax(z)
    o_ref[...] = _softmax(
        jnp.dot(h1, wp_ref[...], preferred_element_type=jnp.float32))

  return pl.pallas_call(
      body,
      grid=(N_NODES // BLK,),
      in_specs=[
          pl.BlockSpec((BLK, D), lambda i: (i, 0)),
          pl.BlockSpec((BLK, D), lambda i: (i, 0)),
          pl.BlockSpec((D, D), lambda i: (0, 0)),
          pl.BlockSpec((D, D), lambda i: (0, 0)),
          pl.BlockSpec((D, D), lambda i: (0, 0)),
      ],
      out_specs=pl.BlockSpec((BLK, D), lambda i: (i, 0)),
      out_shape=jax.ShapeDtypeStruct((N_NODES, D), jnp.float32),
  )(agg, h, w_a, w_h, w_pool)


def kernel(x, edge_index, W_input, W_layer0, W_layer1, W_pool):
  src2, dst = _prep_indices(edge_index.astype(jnp.int32))
  src2 = src2.reshape(NC * NS, NCHUNK, CH)
  dst = dst.reshape(NS, NCHUNK, CH)
  zeros = jnp.zeros((CH, DH), jnp.float32)
  out_idx = jnp.asarray(_OUT_IDX)

  h = _tc_input(x, W_input)
  agg = _segment_sum_sc(
      h.reshape(NC * N_NODES, DH), src2, dst, zeros, out_idx)
  h = _tc_layer(agg.reshape(N_PAD, D), h, W_layer0[:D], W_layer0[D:])
  agg = _segment_sum_sc(
      h.reshape(NC * N_NODES, DH), src2, dst, zeros, out_idx)
  return _tc_layer_pool(
      agg.reshape(N_PAD, D), h, W_layer1[:D], W_layer1[D:], W_pool)
